# Initial kernel scaffold; baseline (speedup 1.0000x reference)
import jax
import jax.numpy as jnp
from jax.experimental import pallas as pl


def kernel(x, edge_index, edge_attr, batch, enc_params, conv_params, dec_params):
    raise NotImplementedError("write your pallas kernel here")


# speedup vs baseline: 4.1796x; 4.1796x over previous
import functools

import jax
import jax.numpy as jnp
from jax.experimental import pallas as pl
from jax.experimental.pallas import tpu as pltpu

N = 10000
IN_CH = 256
HID = 512
MLP_H = 64
OUT_CH = 3
N_GRAPHS = 16
N_CONVS = 3

BN = 1000


def _ln(h, g, b):
    mu = jnp.mean(h, axis=-1, keepdims=True)
    var = jnp.var(h, axis=-1, keepdims=True)
    return (h - mu) * jax.lax.rsqrt(var + 1e-5) * g + b


def _dot(a, b):
    return jax.lax.dot(a, b, preferred_element_type=jnp.float32)


def _mlp(x, W0, b0, W1, b1, W2, b2, g, bt, W3, b3):
    h = jax.nn.relu(_dot(x, W0) + b0)
    h = jax.nn.relu(_dot(h, W1) + b1)
    h = jax.nn.relu(_dot(h, W2) + b2)
    h = _ln(h, g, bt)
    return _dot(h, W3) + b3


def _fused_kernel(n_blocks, x_ref, batch_ref, *refs):
    enc = [r[...] for r in refs[0:10]]
    conv = [[r[...] for r in refs[10 + 10 * c:20 + 10 * c]] for c in range(3)]
    dec = [refs[40 + 8 * d:48 + 8 * d] for d in range(3)]
    out_ref = refs[64]
    acc_ref = refs[65]

    i = pl.program_id(0)

    h = _mlp(x_ref[...], *enc)
    for c in range(3):
        h = h + _mlp(h, *conv[c])

    b = batch_ref[0]
    oh = (jax.lax.broadcasted_iota(jnp.int32, (N_GRAPHS, h.shape[0]), 0)
          == b).astype(jnp.float32)
    part = _dot(oh, h)

    @pl.when(i == 0)
    def _():
        acc_ref[...] = part

    @pl.when(i > 0)
    def _():
        acc_ref[...] = acc_ref[...] + part

    @pl.when(i == n_blocks - 1)
    def _():
        pooled = acc_ref[...]
        cols = []
        for d in range(3):
            g0, t0, W0, b0, g1, t1, w1, b1 = (r[...] for r in dec[d])
            y = jax.nn.relu(_dot(_ln(pooled, g0, t0), W0) + b0)
            y = _ln(y, g1, t1)
            c = jax.nn.relu(jnp.sum(y * w1, axis=1, keepdims=True) + b1)
            cols.append(c)
        out_ref[...] = jnp.concatenate(cols, axis=1)


def kernel(x, edge_index, edge_attr, batch, enc_params, conv_params, dec_params):
    del edge_index, edge_attr
    n_blocks = N // BN

    def prep_mlp(p):
        W0, b0, W1, b1, W2, b2, g, bt, W3, b3 = p
        return [W0, b0.reshape(1, -1), W1, b1.reshape(1, -1),
                W2, b2.reshape(1, -1), g.reshape(1, -1), bt.reshape(1, -1),
                W3, b3.reshape(1, -1)]

    def prep_dec(p):
        g0, t0, W0, b0, g1, t1, W1, b1 = p
        return [g0.reshape(1, -1), t0.reshape(1, -1), W0, b0.reshape(1, -1),
                g1.reshape(1, -1), t1.reshape(1, -1),
                W1.reshape(1, -1),
                b1.reshape(1, -1)]

    weights = []
    weights += prep_mlp(enc_params)
    for (_p1, p2) in conv_params:
        weights += prep_mlp(p2)
    for dp in dec_params:
        weights += prep_dec(dp)

    batch3 = batch.reshape(n_blocks, 1, BN)

    w_specs = [pl.BlockSpec(w.shape, lambda i: (0, 0)) for w in weights]
    in_specs = [
        pl.BlockSpec((BN, IN_CH), lambda i: (i, 0)),
        pl.BlockSpec((1, 1, BN), lambda i: (i, 0, 0)),
    ] + w_specs

    out = pl.pallas_call(
        functools.partial(_fused_kernel, n_blocks),
        grid=(n_blocks,),
        in_specs=in_specs,
        out_specs=pl.BlockSpec((N_GRAPHS, OUT_CH), lambda i: (0, 0)),
        out_shape=jax.ShapeDtypeStruct((N_GRAPHS, OUT_CH), jnp.float32),
        scratch_shapes=[pltpu.VMEM((N_GRAPHS, HID), jnp.float32)],
    )(x, batch3, *weights)
    return out
